# baseline (speedup 1.0000x reference)
import functools

import jax
import jax.numpy as jnp
from jax import lax
from jax.experimental import pallas as pl
from jax.experimental.pallas import tpu as pltpu
from jax.experimental.pallas import tpu_sc as plsc

BATCH = 16384
FIELDS = 26
EMBED_DIM = 32

_B = BATCH * FIELDS
_NW = 32
_CHUNK = 512
_NBG = BATCH // _CHUNK
_NBLK = FIELDS * _NBG
_BLK_PER_W = _NBLK // _NW
_OUT_WORDS = FIELDS * EMBED_DIM * BATCH

_mesh = plsc.VectorSubcoreMesh(core_axis_name="c", subcore_axis_name="s")


@functools.partial(
    pl.kernel,
    mesh=_mesh,
    compiler_params=pltpu.CompilerParams(
        use_tc_tiling_on_sc=False, needs_layout_passes=False),
    out_type=jax.ShapeDtypeStruct((_OUT_WORDS,), jnp.float32),
    scratch_types=[
        tuple(pltpu.VMEM((_CHUNK,), jnp.int32) for _ in range(2)),
        tuple(pltpu.VMEM((_CHUNK, EMBED_DIM), jnp.float32) for _ in range(2)),
        tuple(pltpu.VMEM((_CHUNK * EMBED_DIM,), jnp.float32) for _ in range(2)),
        tuple(pltpu.SemaphoreType.DMA for _ in range(2)),
        tuple(pltpu.SemaphoreType.DMA for _ in range(2)),
    ],
)
def _gather_all(idx_hbm, table_hbm, out_hbm, idx_v, gbuf, stage, gsem, wsem):
    wid = lax.axis_index("s") * 2 + lax.axis_index("c")
    blk0 = wid * _BLK_PER_W

    iota = jax.lax.iota(jnp.int32, 16)
    jvecs = [iota + (j * 16) for j in range(8)]

    def idx_off(beta):
        f = beta >> 5
        bg = beta & 31
        return f * BATCH + bg * _CHUNK

    def start_gather(beta, p):
        pltpu.sync_copy(idx_hbm.at[pl.ds(idx_off(beta), _CHUNK)], idx_v[p])
        pltpu.async_copy(table_hbm.at[idx_v[p]], gbuf[p], gsem[p])

    def wait_gather(p):
        pltpu.make_async_copy(
            table_hbm.at[idx_v[p]], gbuf[p], gsem[p]).wait()

    def block_body(beta, p):
        g = gbuf[p]
        st = stage[p]

        @pl.loop(0, 16)
        def _perm(r):
            dt = r >> 2
            bt = r & 3
            btv = bt * 128 + jnp.zeros((16,), jnp.int32)
            dbase = dt * 8
            for ds in range(8):
                cvec = dbase + ds + jnp.zeros((16,), jnp.int32)
                for j in range(8):
                    rvec = jvecs[j] + btv
                    v = plsc.load_gather(g, [rvec, cvec])
                    st[pl.ds(r * 1024 + ds * 128 + j * 16, 16)] = v

        f = beta >> 5
        bg = beta & 31
        for dt in range(4):
            dst = f * 524288 + dt * 131072 + bg * 4096
            pltpu.async_copy(
                st.at[pl.ds(dt * 4096, 4096)],
                out_hbm.at[pl.ds(dst, 4096)],
                wsem[p])

    def wait_writes(p):
        for dt in range(4):
            pltpu.make_async_copy(
                stage[p].at[pl.ds(dt * 4096, 4096)],
                out_hbm.at[pl.ds(dt * 4096, 4096)],
                wsem[p]).wait()

    start_gather(blk0, 0)

    @pl.loop(0, _BLK_PER_W)
    def _blocks(t):
        even = lax.rem(t, 2) == 0

        @pl.when(t + 1 < _BLK_PER_W)
        def _():
            @pl.when(even)
            def _():
                start_gather(blk0 + t + 1, 1)

            @pl.when(jnp.logical_not(even))
            def _():
                start_gather(blk0 + t + 1, 0)

        @pl.when(even)
        def _():
            wait_gather(0)

            @pl.when(t >= 2)
            def _():
                wait_writes(0)
            block_body(blk0 + t, 0)

        @pl.when(jnp.logical_not(even))
        def _():
            wait_gather(1)

            @pl.when(t >= 2)
            def _():
                wait_writes(1)
            block_body(blk0 + t, 1)

    wait_writes(0)
    wait_writes(1)


def kernel(cat_idx, table):
    idx_flat = cat_idx.T.reshape(_B).astype(jnp.int32)
    out = _gather_all(idx_flat, table)
    out6 = out.reshape(FIELDS, 4, 128, 8, 128)
    return out6.transpose(2, 4, 0, 1, 3).reshape(BATCH, FIELDS, EMBED_DIM)


# speedup vs baseline: 1.1479x; 1.1479x over previous
import functools

import jax
import jax.numpy as jnp
from jax import lax
from jax.experimental import pallas as pl
from jax.experimental.pallas import tpu as pltpu
from jax.experimental.pallas import tpu_sc as plsc

BATCH = 16384
FIELDS = 26
EMBED_DIM = 32

_B = BATCH * FIELDS
_NW = 32
_CHUNK = 256
_NBG = BATCH // _CHUNK
_NBLK = FIELDS * _NBG
_BLK_PER_W = _NBLK // _NW
_OUT_WORDS = FIELDS * EMBED_DIM * BATCH

_mesh = plsc.VectorSubcoreMesh(core_axis_name="c", subcore_axis_name="s")


@functools.partial(
    pl.kernel,
    mesh=_mesh,
    compiler_params=pltpu.CompilerParams(
        use_tc_tiling_on_sc=False, needs_layout_passes=False),
    out_type=jax.ShapeDtypeStruct((_OUT_WORDS,), jnp.float32),
    scratch_types=[
        tuple(pltpu.VMEM((_CHUNK,), jnp.int32) for _ in range(2)),
        tuple(pltpu.VMEM((_CHUNK,), jnp.int32) for _ in range(2)),
        tuple(pltpu.VMEM((_CHUNK,), jnp.int32) for _ in range(2)),
        tuple(pltpu.VMEM((_CHUNK, 128), jnp.float32) for _ in range(2)),
        tuple(pltpu.VMEM((_CHUNK * EMBED_DIM,), jnp.float32) for _ in range(2)),
        tuple(pltpu.SemaphoreType.DMA for _ in range(2)),
        tuple(pltpu.SemaphoreType.DMA for _ in range(2)),
    ],
)
def _gather_all(idx_hbm, table_hbm, out_hbm,
                idx_v, gidx, colb, gbuf, stage, gsem, wsem):
    wid = lax.axis_index("s") * 2 + lax.axis_index("c")
    blk0 = wid * _BLK_PER_W

    iota = jax.lax.iota(jnp.int32, 16)

    def idx_off(beta):
        f = beta >> 6
        bg = beta & 63
        return f * BATCH + bg * _CHUNK

    def start_gather(beta, p):
        pltpu.sync_copy(idx_hbm.at[pl.ds(idx_off(beta), _CHUNK)], idx_v[p])
        for j in range(_CHUNK // 16):
            v = idx_v[p][pl.ds(j * 16, 16)]
            gidx[p][pl.ds(j * 16, 16)] = v >> 2
            colb[p][pl.ds(j * 16, 16)] = (v & 3) * 32
        pltpu.async_copy(table_hbm.at[gidx[p]], gbuf[p], gsem[p])

    def wait_gather(p):
        pltpu.make_async_copy(
            table_hbm.at[gidx[p]], gbuf[p], gsem[p]).wait()

    def block_body(beta, p):
        g = gbuf[p]
        st = stage[p]
        cb = colb[p]

        @plsc.parallel_loop(0, EMBED_DIM, unroll=4)
        def _perm(d):
            dbase = (d >> 3) * 2048 + (d & 7) * 128
            for j in range(16):
                rvec = j * 16 + iota
                cvec = cb[pl.ds(j * 16, 16)] + d
                v = plsc.load_gather(g, [rvec, cvec])
                off = (j >> 3) * 1024 + (j & 7) * 16
                st[pl.ds(dbase + off, 16)] = v

        f = beta >> 6
        bg = beta & 63
        for dt in range(4):
            dst = f * 524288 + dt * 131072 + bg * 2048
            pltpu.async_copy(
                st.at[pl.ds(dt * 2048, 2048)],
                out_hbm.at[pl.ds(dst, 2048)],
                wsem[p])

    def wait_writes(p):
        for dt in range(4):
            pltpu.make_async_copy(
                stage[p].at[pl.ds(dt * 2048, 2048)],
                out_hbm.at[pl.ds(dt * 2048, 2048)],
                wsem[p]).wait()

    start_gather(blk0, 0)

    @pl.loop(0, _BLK_PER_W, step=2)
    def _blocks(t):
        for p in range(2):
            b = t + p

            @pl.when(b + 1 < _BLK_PER_W)
            def _():
                start_gather(blk0 + b + 1, 1 - p)

            wait_gather(p)

            @pl.when(b >= 2)
            def _():
                wait_writes(p)

            block_body(blk0 + b, p)

    wait_writes(0)
    wait_writes(1)


def kernel(cat_idx, table):
    idx_flat = cat_idx.T.reshape(_B).astype(jnp.int32)
    table28 = table.reshape(250000, 128)
    out = _gather_all(idx_flat, table28)
    out6 = out.reshape(FIELDS, 4, 128, 8, 128)
    return out6.transpose(2, 4, 0, 1, 3).reshape(BATCH, FIELDS, EMBED_DIM)


# speedup vs baseline: 1.2007x; 1.0460x over previous
import functools

import jax
import jax.numpy as jnp
from jax import lax
from jax.experimental import pallas as pl
from jax.experimental.pallas import tpu as pltpu
from jax.experimental.pallas import tpu_sc as plsc

BATCH = 16384
FIELDS = 26
EMBED_DIM = 32

_B = BATCH * FIELDS
_NW = 32
_CHUNK = 256
_NBG = BATCH // _CHUNK
_NBLK = FIELDS * _NBG
_BLK_PER_W = _NBLK // _NW
_OUT_WORDS = FIELDS * EMBED_DIM * BATCH

_mesh = plsc.VectorSubcoreMesh(core_axis_name="c", subcore_axis_name="s")


@functools.partial(
    pl.kernel,
    mesh=_mesh,
    compiler_params=pltpu.CompilerParams(
        use_tc_tiling_on_sc=False, needs_layout_passes=False),
    out_type=jax.ShapeDtypeStruct((_OUT_WORDS,), jnp.float32),
    scratch_types=[
        tuple(pltpu.VMEM((_CHUNK,), jnp.int32) for _ in range(2)),
        tuple(pltpu.VMEM((_CHUNK, EMBED_DIM), jnp.float32) for _ in range(2)),
        tuple(pltpu.VMEM((_CHUNK * EMBED_DIM,), jnp.float32) for _ in range(2)),
        tuple(pltpu.SemaphoreType.DMA for _ in range(2)),
        tuple(pltpu.SemaphoreType.DMA for _ in range(2)),
    ],
)
def _gather_all(idx_hbm, table_hbm, out_hbm,
                idx_v, gbuf, stage, gsem, wsem):
    wid = lax.axis_index("s") * 2 + lax.axis_index("c")
    blk0 = wid * _BLK_PER_W

    iota = jax.lax.iota(jnp.int32, 16)

    def idx_off(beta):
        f = beta >> 6
        bg = beta & 63
        return f * BATCH + bg * _CHUNK

    def start_gather(beta, p):
        pltpu.sync_copy(idx_hbm.at[pl.ds(idx_off(beta), _CHUNK)], idx_v[p])
        pltpu.async_copy(table_hbm.at[idx_v[p]], gbuf[p], gsem[p])

    def wait_gather(p):
        pltpu.make_async_copy(
            table_hbm.at[idx_v[p]], gbuf[p], gsem[p]).wait()

    def block_body(beta, p):
        g = gbuf[p]
        st = stage[p]

        @plsc.parallel_loop(0, EMBED_DIM, unroll=4)
        def _perm(d):
            dbase = (d >> 3) * 2048 + (d & 7) * 128
            cvec = d + jnp.zeros((16,), jnp.int32)
            for j in range(16):
                rvec = j * 16 + iota
                v = plsc.load_gather(g, [rvec, cvec])
                off = (j >> 3) * 1024 + (j & 7) * 16
                st[pl.ds(dbase + off, 16)] = v

        f = beta >> 6
        bg = beta & 63
        for dt in range(4):
            dst = f * 524288 + dt * 131072 + bg * 2048
            pltpu.async_copy(
                st.at[pl.ds(dt * 2048, 2048)],
                out_hbm.at[pl.ds(dst, 2048)],
                wsem[p])

    def wait_writes(p):
        for dt in range(4):
            pltpu.make_async_copy(
                stage[p].at[pl.ds(dt * 2048, 2048)],
                out_hbm.at[pl.ds(dt * 2048, 2048)],
                wsem[p]).wait()

    start_gather(blk0, 0)

    @pl.loop(0, _BLK_PER_W, step=2)
    def _blocks(t):
        for p in range(2):
            b = t + p

            @pl.when(b + 1 < _BLK_PER_W)
            def _():
                start_gather(blk0 + b + 1, 1 - p)

            wait_gather(p)

            @pl.when(b >= 2)
            def _():
                wait_writes(p)

            block_body(blk0 + b, p)

    wait_writes(0)
    wait_writes(1)


def kernel(cat_idx, table):
    idx_flat = cat_idx.T.reshape(_B).astype(jnp.int32)
    out = _gather_all(idx_flat, table)
    out6 = out.reshape(FIELDS, 4, 128, 8, 128)
    return out6.transpose(2, 4, 0, 1, 3).reshape(BATCH, FIELDS, EMBED_DIM)
